# baseline (speedup 1.0000x reference)
import ml_dtypes
import numpy as np
import jax
import jax.numpy as jnp
from jax.experimental import pallas as pl
from jax.experimental.pallas import tpu as pltpu

Q = 4096
N = 100000
D = 128
KTOP = 100
NEG = np.float32(-3.0e38)
BIG = np.float32(3.0e38)

NPAD = 102400
SUB = 16
NSUBC = NPAD // SUB
CHUNK = 256
NCHUNK = NPAD // CHUNK
REAL_CHUNKS = (N + CHUNK - 1) // CHUNK
CIDK = 128
SLOTS = 256
NCHUNK_P = 512

ROW_TILE = 256
COL_TILE = 4096


def _fiota(shape, dim):
    return jax.lax.broadcasted_iota(jnp.int32, shape, dim).astype(jnp.float32)


def _runmax16(z, width):
    for d in (1, 2, 4, 8):
        z = jnp.maximum(z, pltpu.roll(z, width - d, 1))
    return z


def _excl_slots(m, b_ref, u128_ref, eb_ref):
    mf = m.astype(jnp.float32)
    lane = jax.lax.broadcasted_iota(jnp.int32, m.shape, 1) % 32
    z = jnp.where(lane >= 1, pltpu.roll(mf, 1, 1), np.float32(0.0))
    for d in (1, 2, 4, 8, 16):
        z = z + jnp.where(lane >= d, pltpu.roll(z, d, 1), np.float32(0.0))
    bs = jnp.dot(m.astype(jnp.bfloat16), b_ref[...],
                 preferred_element_type=jnp.float32)
    off = jnp.dot(bs.astype(jnp.bfloat16), u128_ref[...],
                  preferred_element_type=jnp.float32)
    return z + jnp.dot(off, eb_ref[...], preferred_element_type=jnp.float32,
                       precision=jax.lax.Precision.HIGHEST)


def _scan_consts(width):
    nb = width // 32
    b = np.zeros((width, 128), np.float32)
    b[np.arange(width), np.arange(width) // 32] = 1.0
    u128 = np.triu(np.ones((128, 128), np.float32), 1)
    eb = np.zeros((128, width), np.float32)
    eb[np.arange(width) // 32, np.arange(width)] = 1.0
    assert nb <= 128
    return (b.astype(ml_dtypes.bfloat16), u128.astype(ml_dtypes.bfloat16), eb)


def _k1_body(a_ref, b_ref, s16_ref, sims_ref, smx_ref):
    j = pl.program_id(0)
    y = jnp.dot(a_ref[...], b_ref[...].T, preferred_element_type=jnp.float32)
    col = j * COL_TILE + jax.lax.broadcasted_iota(jnp.int32, (ROW_TILE, COL_TILE), 1)
    y = jnp.where(col < N, y, NEG)
    sims_ref[...] = y
    z = _runmax16(y, COL_TILE)
    smx_ref[...] = jnp.dot(z, s16_ref[...], preferred_element_type=jnp.float32,
                           precision=jax.lax.Precision.HIGHEST)


def _k1(image_embs, text_pad):
    s16 = np.zeros((COL_TILE, COL_TILE // SUB), np.float32)
    s16[np.arange(0, COL_TILE, SUB), np.arange(COL_TILE // SUB)] = 1.0
    return pl.pallas_call(
        _k1_body,
        grid=(NPAD // COL_TILE, Q // ROW_TILE),
        in_specs=[
            pl.BlockSpec((ROW_TILE, D), lambda j, i: (i, 0)),
            pl.BlockSpec((COL_TILE, D), lambda j, i: (j, 0)),
            pl.BlockSpec((COL_TILE, COL_TILE // SUB), lambda j, i: (0, 0)),
        ],
        out_specs=[
            pl.BlockSpec((ROW_TILE, COL_TILE), lambda j, i: (i, j)),
            pl.BlockSpec((ROW_TILE, COL_TILE // SUB), lambda j, i: (i, j)),
        ],
        out_shape=[
            jax.ShapeDtypeStruct((Q, NPAD), jnp.float32),
            jax.ShapeDtypeStruct((Q, NSUBC), jnp.float32),
        ],
    )(image_embs, text_pad, s16)


R2 = 64

def _k2_body(smx_ref, sc_ref, u5_ref, cidf_ref, t_ref):
    pid = pl.program_id(0)
    s = smx_ref[...]
    z = _runmax16(s, NSUBC)
    cm = jnp.dot(z, sc_ref[...], preferred_element_type=jnp.float32,
                 precision=jax.lax.Precision.HIGHEST)
    uvec = jax.lax.broadcasted_iota(jnp.int32, (R2, NCHUNK_P), 1)
    cm = jnp.where(uvec < NCHUNK, cm, NEG)

    bits = jax.lax.bitcast_convert_type(cm, jnp.uint32)
    neg = bits >= np.uint32(0x80000000)
    keys = jnp.where(neg, ~bits, bits | np.uint32(0x80000000))

    def bisect(i, pfx):
        cand = pfx | jax.lax.shift_left(
            np.uint32(1), (31 - i).astype(jnp.uint32))
        cnt = jnp.sum((keys >= cand).astype(jnp.float32), axis=1, keepdims=True)
        return jnp.where(cnt >= np.float32(KTOP), cand, pfx)

    pfx = jax.lax.fori_loop(0, 32, bisect, jnp.zeros((R2, 1), jnp.uint32))
    kneg = pfx < np.uint32(0x80000000)
    kbits = jnp.where(kneg, ~pfx, pfx & np.uint32(0x7FFFFFFF))
    t = jax.lax.bitcast_convert_type(kbits, jnp.float32)

    mask = cm >= t
    slots = jnp.dot(mask.astype(jnp.bfloat16), u5_ref[...],
                    preferred_element_type=jnp.float32)
    count = jnp.sum(mask.astype(jnp.float32), axis=1, keepdims=True)

    gval = _fiota((R2, NCHUNK_P), 1)
    parts = []
    for s0 in range(0, CIDK, 64):
        sv = np.float32(s0) + _fiota((R2, 64, NCHUNK_P), 1)
        oh = (mask[:, None, :] & (slots[:, None, :] == sv)).astype(jnp.float32)
        parts.append(jnp.sum(oh * gval[:, None, :], axis=2))
    cid = jnp.concatenate(parts, axis=1)

    svec = _fiota((R2, CIDK), 1)
    smod = (jax.lax.broadcasted_iota(jnp.int32, (R2, CIDK), 1) % 9).astype(jnp.float32)
    dummy = np.float32(REAL_CHUNKS) + smod
    cidl = jnp.where(svec < count, cid, dummy)
    row = pid * R2 + _fiota((R2, 1), 0)
    cidf_ref[...] = (cidl + row * NCHUNK).astype(jnp.int32)
    t_ref[...] = jnp.broadcast_to(t, (R2, 128))


def _k2(smx):
    sc = np.zeros((NSUBC, NCHUNK_P), np.float32)
    sc[np.arange(0, NSUBC, SUB), np.arange(NCHUNK)] = 1.0
    u5 = np.triu(np.ones((NCHUNK_P, NCHUNK_P), np.float32), 1).astype(ml_dtypes.bfloat16)
    return pl.pallas_call(
        _k2_body,
        grid=(Q // R2,),
        in_specs=[
            pl.BlockSpec((R2, NSUBC), lambda r: (r, 0)),
            pl.BlockSpec((NSUBC, NCHUNK_P), lambda r: (0, 0)),
            pl.BlockSpec((NCHUNK_P, NCHUNK_P), lambda r: (0, 0)),
        ],
        out_specs=[
            pl.BlockSpec((R2, CIDK), lambda r: (r, 0)),
            pl.BlockSpec((R2, 128), lambda r: (r, 0)),
        ],
        out_shape=[
            jax.ShapeDtypeStruct((Q, CIDK), jnp.int32),
            jax.ShapeDtypeStruct((Q, 128), jnp.float32),
        ],
    )(smx, sc, u5)


R4 = 8
W4 = CIDK * SUB

def _k4_body(smxg_ref, cidf_ref, t_ref, b_ref, u128_ref, eb_ref, sgf_ref, sgl_ref):
    pid = pl.program_id(0)
    g = smxg_ref[...]
    t = t_ref[...][:, 0:1]
    m = g >= t

    slots = _excl_slots(m, b_ref, u128_ref, eb_ref)
    count = jnp.sum(m.astype(jnp.float32), axis=1, keepdims=True)

    row = pid * R4 + _fiota((R4, 1), 0)
    cc = jax.lax.broadcasted_iota(jnp.int32, (CIDK, W4), 0)
    jj = jax.lax.broadcasted_iota(jnp.int32, (CIDK, W4), 1)
    expand = (jj // SUB == cc).astype(jnp.float32)
    cid_local = cidf_ref[...].astype(jnp.float32) - row * NCHUNK
    cid_full = jnp.dot(cid_local, expand, preferred_element_type=jnp.float32,
                       precision=jax.lax.Precision.HIGHEST)
    u = jax.lax.broadcasted_iota(jnp.int32, (R4, W4), 1) % SUB
    sg_full = cid_full * SUB + u.astype(jnp.float32)

    parts = []
    for s0 in range(0, SLOTS, 64):
        sv = np.float32(s0) + _fiota((R4, 64, W4), 1)
        oh = (m[:, None, :] & (slots[:, None, :] == sv)).astype(jnp.float32)
        parts.append(jnp.sum(oh * sg_full[:, None, :], axis=2))
    sg = jnp.concatenate(parts, axis=1)

    svec = _fiota((R4, SLOTS), 1)
    smod = (jax.lax.broadcasted_iota(jnp.int32, (R4, SLOTS), 1) % 128).astype(jnp.float32)
    dummy = np.float32(N // SUB) + smod
    sgl = jnp.where(svec < count, sg, dummy)
    sgl_ref[...] = sgl.astype(jnp.int32)
    sgf_ref[...] = sgl.astype(jnp.int32) + (row.astype(jnp.int32) * NSUBC)


def _k4(smxg, cidf, t):
    b, u128, eb = _scan_consts(W4)
    return pl.pallas_call(
        _k4_body,
        grid=(Q // R4,),
        in_specs=[
            pl.BlockSpec((R4, W4), lambda r: (r, 0)),
            pl.BlockSpec((R4, CIDK), lambda r: (r, 0)),
            pl.BlockSpec((R4, 128), lambda r: (r, 0)),
            pl.BlockSpec((W4, 128), lambda r: (0, 0)),
            pl.BlockSpec((128, 128), lambda r: (0, 0)),
            pl.BlockSpec((128, W4), lambda r: (0, 0)),
        ],
        out_specs=[
            pl.BlockSpec((R4, SLOTS), lambda r: (r, 0)),
            pl.BlockSpec((R4, SLOTS), lambda r: (r, 0)),
        ],
        out_shape=[
            jax.ShapeDtypeStruct((Q, SLOTS), jnp.int32),
            jax.ShapeDtypeStruct((Q, SLOTS), jnp.int32),
        ],
    )(smxg, cidf, t, b, u128, eb)


R6 = 8
W6 = SLOTS * SUB

def _k6_body(vg_ref, sgl_ref, t_ref, b_ref, u128_ref, eb_ref, e_ref, out_ref):
    v = vg_ref[...]
    t = t_ref[...][:, 0:1]
    m = v >= t

    slots = _excl_slots(m, b_ref, u128_ref, eb_ref)
    sg_full = jnp.dot(sgl_ref[...].astype(jnp.float32), e_ref[...],
                      preferred_element_type=jnp.float32,
                      precision=jax.lax.Precision.HIGHEST)
    u = jax.lax.broadcasted_iota(jnp.int32, (R6, W6), 1) % SUB
    col = sg_full * SUB + u.astype(jnp.float32)

    vl_p, cl_p, vs_p, cs_p, fl_p, fs_p = [], [], [], [], [], []
    SBLK = 16
    for s0 in range(0, SLOTS, SBLK):
        sv = np.float32(s0) + _fiota((R6, SBLK, W6), 1)
        oh = (m[:, None, :] & (slots[:, None, :] == sv)).astype(jnp.float32)
        ohv = oh * v[:, None, :]
        ohc = oh * col[:, None, :]
        vl_p.append(jnp.sum(ohv, axis=2))
        cl_p.append(jnp.sum(ohc, axis=2))
        vs_p.append(jnp.sum(ohv, axis=2, keepdims=True))
        cs_p.append(jnp.sum(ohc, axis=2, keepdims=True))
        fl_p.append(jnp.sum(oh, axis=2))
        fs_p.append(jnp.sum(oh, axis=2, keepdims=True))
    vl = jnp.concatenate(vl_p, axis=1)
    cl = jnp.concatenate(cl_p, axis=1)
    fill = jnp.concatenate(fl_p, axis=1)
    vs = jnp.concatenate(vs_p, axis=1)
    cs = jnp.concatenate(cs_p, axis=1)
    fill3 = jnp.concatenate(fs_p, axis=1)

    svec = _fiota((R6, SLOTS), 1)
    vl = vl + (fill - 1.0) * BIG
    cl = cl + (1.0 - fill) * (np.float32(2.0e6) + svec)
    svec3 = _fiota((R6, SLOTS, 1), 1)
    vs = vs + (fill3 - 1.0) * BIG
    cs = cs + (1.0 - fill3) * (np.float32(2.0e6) + svec3)

    vk = vl[:, None, :]
    ck = cl[:, None, :]
    better = (vk > vs) | ((vk == vs) & (ck < cs))
    rank = jnp.sum(better.astype(jnp.float32), axis=2)

    pv = _fiota((R6, 128, SLOTS), 1)
    ohp = (rank[:, None, :] == pv).astype(jnp.float32)
    out = jnp.sum(ohp * cl[:, None, :], axis=2)
    out_ref[...] = out.astype(jnp.int32)


def _k6(vg, sgl, t):
    b, u128, eb = _scan_consts(W6)
    cc = np.arange(SLOTS, dtype=np.int64)
    jj = np.arange(W6, dtype=np.int64)
    ee = (jj[None, :] // SUB == cc[:, None]).astype(np.float32)
    return pl.pallas_call(
        _k6_body,
        grid=(Q // R6,),
        in_specs=[
            pl.BlockSpec((R6, W6), lambda r: (r, 0)),
            pl.BlockSpec((R6, SLOTS), lambda r: (r, 0)),
            pl.BlockSpec((R6, 128), lambda r: (r, 0)),
            pl.BlockSpec((W6, 128), lambda r: (0, 0)),
            pl.BlockSpec((128, 128), lambda r: (0, 0)),
            pl.BlockSpec((128, W6), lambda r: (0, 0)),
            pl.BlockSpec((SLOTS, W6), lambda r: (0, 0)),
        ],
        out_specs=pl.BlockSpec((R6, 128), lambda r: (r, 0)),
        out_shape=jax.ShapeDtypeStruct((Q, 128), jnp.int32),
    )(vg, sgl, t, b, u128, eb, ee)


def kernel(image_embs, text_embs):
    text_pad = jnp.pad(text_embs, ((0, NPAD - N), (0, 0)))
    sims, smx = _k1(image_embs, text_pad)

    cidf, t = _k2(smx)

    smx_tab = smx.reshape(Q * NCHUNK, SUB)
    smxg = jnp.take(smx_tab, cidf.reshape(-1), axis=0, mode="clip")
    smxg = smxg.reshape(Q, W4)

    sgf, sgl = _k4(smxg, cidf, t)

    sims_tab = sims.reshape(Q * NSUBC, SUB)
    vg = jnp.take(sims_tab, sgf.reshape(-1), axis=0, mode="clip")
    vg = vg.reshape(Q, W6)

    idx = _k6(vg, sgl, t)
    return idx[:, :KTOP]


# speedup vs baseline: 1.4844x; 1.4844x over previous
import ml_dtypes
import numpy as np
import jax
import jax.numpy as jnp
from jax.experimental import pallas as pl
from jax.experimental.pallas import tpu as pltpu

Q = 4096
N = 100000
D = 128
KTOP = 100
NEG = np.float32(-3.0e38)
BIG = np.float32(3.0e38)

NPAD = 102400
SUB = 16
NSUBC = NPAD // SUB
CHUNK = 256
NCHUNK = NPAD // CHUNK
REAL_CHUNKS = (N + CHUNK - 1) // CHUNK
CIDK = 128
SLOTS = 256
NCHUNK_P = 512

ROW_TILE = 256
COL_TILE = 4096


def _fiota(shape, dim):
    return jax.lax.broadcasted_iota(jnp.int32, shape, dim).astype(jnp.float32)


def _runmax16(z, width):
    for d in (1, 2, 4, 8):
        z = jnp.maximum(z, pltpu.roll(z, width - d, 1))
    return z


def _excl_slots(m, b_ref, u128_ref, eb_ref):
    mf = m.astype(jnp.float32)
    lane = jax.lax.broadcasted_iota(jnp.int32, m.shape, 1) % 32
    z = jnp.where(lane >= 1, pltpu.roll(mf, 1, 1), np.float32(0.0))
    for d in (1, 2, 4, 8, 16):
        z = z + jnp.where(lane >= d, pltpu.roll(z, d, 1), np.float32(0.0))
    bs = jnp.dot(m.astype(jnp.bfloat16), b_ref[...],
                 preferred_element_type=jnp.float32)
    off = jnp.dot(bs.astype(jnp.bfloat16), u128_ref[...],
                  preferred_element_type=jnp.float32)
    return z + jnp.dot(off, eb_ref[...], preferred_element_type=jnp.float32,
                       precision=jax.lax.Precision.HIGHEST)


def _scan_consts(width):
    nb = width // 32
    b = np.zeros((width, 128), np.float32)
    b[np.arange(width), np.arange(width) // 32] = 1.0
    u128 = np.triu(np.ones((128, 128), np.float32), 1)
    eb = np.zeros((128, width), np.float32)
    eb[np.arange(width) // 32, np.arange(width)] = 1.0
    assert nb <= 128
    return (b.astype(ml_dtypes.bfloat16), u128.astype(ml_dtypes.bfloat16), eb)


def _k1_body(a_ref, b_ref, s16_ref, sims_ref, smx_ref):
    j = pl.program_id(0)
    y = jnp.dot(a_ref[...], b_ref[...].T, preferred_element_type=jnp.float32)
    col = j * COL_TILE + jax.lax.broadcasted_iota(jnp.int32, (ROW_TILE, COL_TILE), 1)
    y = jnp.where(col < N, y, NEG)
    sims_ref[...] = y
    z = _runmax16(y, COL_TILE)
    smx_ref[...] = jnp.dot(z, s16_ref[...], preferred_element_type=jnp.float32,
                           precision=jax.lax.Precision.HIGHEST)


def _k1(image_embs, text_pad):
    s16 = np.zeros((COL_TILE, COL_TILE // SUB), np.float32)
    s16[np.arange(0, COL_TILE, SUB), np.arange(COL_TILE // SUB)] = 1.0
    return pl.pallas_call(
        _k1_body,
        grid=(NPAD // COL_TILE, Q // ROW_TILE),
        in_specs=[
            pl.BlockSpec((ROW_TILE, D), lambda j, i: (i, 0)),
            pl.BlockSpec((COL_TILE, D), lambda j, i: (j, 0)),
            pl.BlockSpec((COL_TILE, COL_TILE // SUB), lambda j, i: (0, 0)),
        ],
        out_specs=[
            pl.BlockSpec((ROW_TILE, COL_TILE), lambda j, i: (i, j)),
            pl.BlockSpec((ROW_TILE, COL_TILE // SUB), lambda j, i: (i, j)),
        ],
        out_shape=[
            jax.ShapeDtypeStruct((Q, NPAD), jnp.float32),
            jax.ShapeDtypeStruct((Q, NSUBC), jnp.float32),
        ],
    )(image_embs, text_pad, s16)


R2 = 64

def _k2_body(smx_ref, sc_ref, u5_ref, cidf_ref, t_ref):
    pid = pl.program_id(0)
    s = smx_ref[...]
    z = _runmax16(s, NSUBC)
    cm = jnp.dot(z, sc_ref[...], preferred_element_type=jnp.float32,
                 precision=jax.lax.Precision.HIGHEST)
    uvec = jax.lax.broadcasted_iota(jnp.int32, (R2, NCHUNK_P), 1)
    cm = jnp.where(uvec < NCHUNK, cm, NEG)

    bits = jax.lax.bitcast_convert_type(cm, jnp.uint32)
    neg = bits >= np.uint32(0x80000000)
    keys = jnp.where(neg, ~bits, bits | np.uint32(0x80000000))

    def bisect(i, pfx):
        cand = pfx | jax.lax.shift_left(
            np.uint32(1), (31 - i).astype(jnp.uint32))
        cnt = jnp.sum((keys >= cand).astype(jnp.float32), axis=1, keepdims=True)
        return jnp.where(cnt >= np.float32(KTOP), cand, pfx)

    pfx = jax.lax.fori_loop(0, 32, bisect, jnp.zeros((R2, 1), jnp.uint32))
    kneg = pfx < np.uint32(0x80000000)
    kbits = jnp.where(kneg, ~pfx, pfx & np.uint32(0x7FFFFFFF))
    t = jax.lax.bitcast_convert_type(kbits, jnp.float32)

    mask = cm >= t
    slots = jnp.dot(mask.astype(jnp.bfloat16), u5_ref[...],
                    preferred_element_type=jnp.float32)
    count = jnp.sum(mask.astype(jnp.float32), axis=1, keepdims=True)

    gval = _fiota((R2, NCHUNK_P), 1)
    parts = []
    for s0 in range(0, CIDK, 64):
        sv = np.float32(s0) + _fiota((R2, 64, NCHUNK_P), 1)
        oh = (mask[:, None, :] & (slots[:, None, :] == sv)).astype(jnp.float32)
        parts.append(jnp.sum(oh * gval[:, None, :], axis=2))
    cid = jnp.concatenate(parts, axis=1)

    svec = _fiota((R2, CIDK), 1)
    smod = (jax.lax.broadcasted_iota(jnp.int32, (R2, CIDK), 1) % 9).astype(jnp.float32)
    dummy = np.float32(REAL_CHUNKS) + smod
    cidl = jnp.where(svec < count, cid, dummy)
    row = pid * R2 + _fiota((R2, 1), 0)
    cidf_ref[...] = (cidl + row * NCHUNK).astype(jnp.int32)
    t_ref[...] = jnp.broadcast_to(t, (R2, 128))


def _k2(smx):
    sc = np.zeros((NSUBC, NCHUNK_P), np.float32)
    sc[np.arange(0, NSUBC, SUB), np.arange(NCHUNK)] = 1.0
    u5 = np.triu(np.ones((NCHUNK_P, NCHUNK_P), np.float32), 1).astype(ml_dtypes.bfloat16)
    return pl.pallas_call(
        _k2_body,
        grid=(Q // R2,),
        in_specs=[
            pl.BlockSpec((R2, NSUBC), lambda r: (r, 0)),
            pl.BlockSpec((NSUBC, NCHUNK_P), lambda r: (0, 0)),
            pl.BlockSpec((NCHUNK_P, NCHUNK_P), lambda r: (0, 0)),
        ],
        out_specs=[
            pl.BlockSpec((R2, CIDK), lambda r: (r, 0)),
            pl.BlockSpec((R2, 128), lambda r: (r, 0)),
        ],
        out_shape=[
            jax.ShapeDtypeStruct((Q, CIDK), jnp.int32),
            jax.ShapeDtypeStruct((Q, 128), jnp.float32),
        ],
    )(smx, sc, u5)


R4 = 8
W4 = CIDK * SUB

def _k4_body(smxg_ref, cidf_ref, t_ref, b_ref, u128_ref, eb_ref, sgf_ref, sgl_ref):
    pid = pl.program_id(0)
    g = smxg_ref[...]
    t = t_ref[...][:, 0:1]
    m = g >= t

    slots = _excl_slots(m, b_ref, u128_ref, eb_ref)
    count = jnp.sum(m.astype(jnp.float32), axis=1, keepdims=True)

    row = pid * R4 + _fiota((R4, 1), 0)
    cc = jax.lax.broadcasted_iota(jnp.int32, (CIDK, W4), 0)
    jj = jax.lax.broadcasted_iota(jnp.int32, (CIDK, W4), 1)
    expand = (jj // SUB == cc).astype(jnp.float32)
    cid_local = cidf_ref[...].astype(jnp.float32) - row * NCHUNK
    cid_full = jnp.dot(cid_local, expand, preferred_element_type=jnp.float32,
                       precision=jax.lax.Precision.HIGHEST)
    u = jax.lax.broadcasted_iota(jnp.int32, (R4, W4), 1) % SUB
    sg_full = cid_full * SUB + u.astype(jnp.float32)

    parts = []
    for s0 in range(0, SLOTS, 64):
        sv = np.float32(s0) + _fiota((R4, 64, W4), 1)
        oh = (m[:, None, :] & (slots[:, None, :] == sv)).astype(jnp.float32)
        parts.append(jnp.sum(oh * sg_full[:, None, :], axis=2))
    sg = jnp.concatenate(parts, axis=1)

    svec = _fiota((R4, SLOTS), 1)
    smod = (jax.lax.broadcasted_iota(jnp.int32, (R4, SLOTS), 1) % 128).astype(jnp.float32)
    dummy = np.float32(N // SUB) + smod
    sgl = jnp.where(svec < count, sg, dummy)
    sgl_ref[...] = sgl.astype(jnp.int32)
    sgf_ref[...] = sgl.astype(jnp.int32) + (row.astype(jnp.int32) * NSUBC)


def _k4(smxg, cidf, t):
    b, u128, eb = _scan_consts(W4)
    return pl.pallas_call(
        _k4_body,
        grid=(Q // R4,),
        in_specs=[
            pl.BlockSpec((R4, W4), lambda r: (r, 0)),
            pl.BlockSpec((R4, CIDK), lambda r: (r, 0)),
            pl.BlockSpec((R4, 128), lambda r: (r, 0)),
            pl.BlockSpec((W4, 128), lambda r: (0, 0)),
            pl.BlockSpec((128, 128), lambda r: (0, 0)),
            pl.BlockSpec((128, W4), lambda r: (0, 0)),
        ],
        out_specs=[
            pl.BlockSpec((R4, SLOTS), lambda r: (r, 0)),
            pl.BlockSpec((R4, SLOTS), lambda r: (r, 0)),
        ],
        out_shape=[
            jax.ShapeDtypeStruct((Q, SLOTS), jnp.int32),
            jax.ShapeDtypeStruct((Q, SLOTS), jnp.int32),
        ],
    )(smxg, cidf, t, b, u128, eb)


R6 = 8
W6 = SLOTS * SUB

def _k6_body(vg_ref, sgl_ref, t_ref, b_ref, u128_ref, eb_ref, e_ref, out_ref):
    v = vg_ref[...]
    t = t_ref[...][:, 0:1]
    m = v >= t

    slots = _excl_slots(m, b_ref, u128_ref, eb_ref)
    count = jnp.sum(m.astype(jnp.float32), axis=1, keepdims=True)
    sg_full = jnp.dot(sgl_ref[...].astype(jnp.float32), e_ref[...],
                      preferred_element_type=jnp.float32,
                      precision=jax.lax.Precision.HIGHEST)
    u = jax.lax.broadcasted_iota(jnp.int32, (R6, W6), 1) % SUB
    col = sg_full * SUB + u.astype(jnp.float32)

    vparts, cparts = [], []
    for s0 in range(0, SLOTS, 64):
        sv = np.float32(s0) + _fiota((R6, 64, W6), 1)
        oh = (m[:, None, :] & (slots[:, None, :] == sv)).astype(jnp.float32)
        vparts.append(jnp.sum(oh * v[:, None, :], axis=2))
        cparts.append(jnp.sum(oh * col[:, None, :], axis=2))
    v256 = jnp.concatenate(vparts, axis=1)
    c256 = jnp.concatenate(cparts, axis=1)

    svec = _fiota((R6, SLOTS), 1)
    valid = svec < count
    v256 = jnp.where(valid, v256, NEG)
    c256 = jnp.where(valid, c256, np.float32(2.0e6) + svec)

    lane = jax.lax.broadcasted_iota(jnp.int32, (R6, 128), 1)

    def extract(p, carry):
        vc, out = carry
        mx = jnp.max(vc, axis=1, keepdims=True)
        ismx = vc == mx
        cmin = jnp.min(jnp.where(ismx, c256, BIG), axis=1, keepdims=True)
        chosen = ismx & (c256 == cmin)
        out = out + jnp.where(lane == p, cmin, np.float32(0.0))
        vc = jnp.where(chosen, NEG, vc)
        return (vc, out)

    _, out = jax.lax.fori_loop(
        0, KTOP, extract, (v256, jnp.zeros((R6, 128), jnp.float32)))
    out_ref[...] = out.astype(jnp.int32)


def _k6(vg, sgl, t):
    b, u128, eb = _scan_consts(W6)
    cc = np.arange(SLOTS, dtype=np.int64)
    jj = np.arange(W6, dtype=np.int64)
    ee = (jj[None, :] // SUB == cc[:, None]).astype(np.float32)
    return pl.pallas_call(
        _k6_body,
        grid=(Q // R6,),
        in_specs=[
            pl.BlockSpec((R6, W6), lambda r: (r, 0)),
            pl.BlockSpec((R6, SLOTS), lambda r: (r, 0)),
            pl.BlockSpec((R6, 128), lambda r: (r, 0)),
            pl.BlockSpec((W6, 128), lambda r: (0, 0)),
            pl.BlockSpec((128, 128), lambda r: (0, 0)),
            pl.BlockSpec((128, W6), lambda r: (0, 0)),
            pl.BlockSpec((SLOTS, W6), lambda r: (0, 0)),
        ],
        out_specs=pl.BlockSpec((R6, 128), lambda r: (r, 0)),
        out_shape=jax.ShapeDtypeStruct((Q, 128), jnp.int32),
    )(vg, sgl, t, b, u128, eb, ee)


def kernel(image_embs, text_embs):
    text_pad = jnp.pad(text_embs, ((0, NPAD - N), (0, 0)))
    sims, smx = _k1(image_embs, text_pad)

    cidf, t = _k2(smx)

    smx_tab = smx.reshape(Q * NCHUNK, SUB)
    smxg = jnp.take(smx_tab, cidf.reshape(-1), axis=0, mode="clip")
    smxg = smxg.reshape(Q, W4)

    sgf, sgl = _k4(smxg, cidf, t)

    sims_tab = sims.reshape(Q * NSUBC, SUB)
    vg = jnp.take(sims_tab, sgf.reshape(-1), axis=0, mode="clip")
    vg = vg.reshape(Q, W6)

    idx = _k6(vg, sgl, t)
    return idx[:, :KTOP]


# speedup vs baseline: 1.9203x; 1.2936x over previous
import ml_dtypes
import numpy as np
import jax
import jax.numpy as jnp
from jax.experimental import pallas as pl
from jax.experimental.pallas import tpu as pltpu

Q = 4096
N = 100000
D = 128
KTOP = 100
NEG = np.float32(-3.0e38)
BIG = np.float32(3.0e38)

NPAD = 102400
SUB = 16
NSUBC = NPAD // SUB
CHUNK = 256
NCHUNK = NPAD // CHUNK
REAL_CHUNKS = (N + CHUNK - 1) // CHUNK
CIDK = 128
SLOTS = 256
NCHUNK_P = 512

ROW_TILE = 256
COL_TILE = 4096


def _fiota(shape, dim):
    return jax.lax.broadcasted_iota(jnp.int32, shape, dim).astype(jnp.float32)


def _runmax16(z, width):
    for d in (1, 2, 4, 8):
        z = jnp.maximum(z, pltpu.roll(z, width - d, 1))
    return z


def _excl_slots(m, b_ref, u128_ref, eb_ref):
    mf = m.astype(jnp.float32)
    lane = jax.lax.broadcasted_iota(jnp.int32, m.shape, 1) % 32
    z = jnp.where(lane >= 1, pltpu.roll(mf, 1, 1), np.float32(0.0))
    for d in (1, 2, 4, 8, 16):
        z = z + jnp.where(lane >= d, pltpu.roll(z, d, 1), np.float32(0.0))
    bs = jnp.dot(m.astype(jnp.bfloat16), b_ref[...],
                 preferred_element_type=jnp.float32)
    off = jnp.dot(bs.astype(jnp.bfloat16), u128_ref[...],
                  preferred_element_type=jnp.float32)
    return z + jnp.dot(off, eb_ref[...], preferred_element_type=jnp.float32,
                       precision=jax.lax.Precision.HIGHEST)


def _scan_consts(width):
    nb = width // 32
    b = np.zeros((width, 128), np.float32)
    b[np.arange(width), np.arange(width) // 32] = 1.0
    u128 = np.triu(np.ones((128, 128), np.float32), 1)
    eb = np.zeros((128, width), np.float32)
    eb[np.arange(width) // 32, np.arange(width)] = 1.0
    assert nb <= 128
    return (b.astype(ml_dtypes.bfloat16), u128.astype(ml_dtypes.bfloat16), eb)


def _k1_body(a_ref, b_ref, s16_ref, sims_ref, smx_ref):
    j = pl.program_id(0)
    y = jnp.dot(a_ref[...], b_ref[...].T, preferred_element_type=jnp.float32)
    col = j * COL_TILE + jax.lax.broadcasted_iota(jnp.int32, (ROW_TILE, COL_TILE), 1)
    y = jnp.where(col < N, y, NEG)
    sims_ref[...] = y
    z = _runmax16(y, COL_TILE)
    smx_ref[...] = jnp.dot(z, s16_ref[...], preferred_element_type=jnp.float32,
                           precision=jax.lax.Precision.HIGHEST)


def _k1(image_embs, text_pad):
    s16 = np.zeros((COL_TILE, COL_TILE // SUB), np.float32)
    s16[np.arange(0, COL_TILE, SUB), np.arange(COL_TILE // SUB)] = 1.0
    return pl.pallas_call(
        _k1_body,
        grid=(NPAD // COL_TILE, Q // ROW_TILE),
        in_specs=[
            pl.BlockSpec((ROW_TILE, D), lambda j, i: (i, 0)),
            pl.BlockSpec((COL_TILE, D), lambda j, i: (j, 0)),
            pl.BlockSpec((COL_TILE, COL_TILE // SUB), lambda j, i: (0, 0)),
        ],
        out_specs=[
            pl.BlockSpec((ROW_TILE, COL_TILE), lambda j, i: (i, j)),
            pl.BlockSpec((ROW_TILE, COL_TILE // SUB), lambda j, i: (i, j)),
        ],
        out_shape=[
            jax.ShapeDtypeStruct((Q, NPAD), jnp.float32),
            jax.ShapeDtypeStruct((Q, NSUBC), jnp.float32),
        ],
    )(image_embs, text_pad, s16)


R2 = 64

def _k2_body(smx_ref, sc_ref, u5_ref, cidf_ref, t_ref):
    pid = pl.program_id(0)
    s = smx_ref[...]
    z = _runmax16(s, NSUBC)
    cm = jnp.dot(z, sc_ref[...], preferred_element_type=jnp.float32,
                 precision=jax.lax.Precision.HIGHEST)
    uvec = jax.lax.broadcasted_iota(jnp.int32, (R2, NCHUNK_P), 1)
    cm = jnp.where(uvec < NCHUNK, cm, NEG)

    bits = jax.lax.bitcast_convert_type(cm, jnp.uint32)
    neg = bits >= np.uint32(0x80000000)
    keys = jnp.where(neg, ~bits, bits | np.uint32(0x80000000))

    def bisect(i, pfx):
        cand = pfx | jax.lax.shift_left(
            np.uint32(1), (31 - i).astype(jnp.uint32))
        cnt = jnp.sum((keys >= cand).astype(jnp.float32), axis=1, keepdims=True)
        return jnp.where(cnt >= np.float32(KTOP), cand, pfx)

    pfx = jax.lax.fori_loop(0, 32, bisect, jnp.zeros((R2, 1), jnp.uint32))
    kneg = pfx < np.uint32(0x80000000)
    kbits = jnp.where(kneg, ~pfx, pfx & np.uint32(0x7FFFFFFF))
    t = jax.lax.bitcast_convert_type(kbits, jnp.float32)

    mask = cm >= t
    slots = jnp.dot(mask.astype(jnp.bfloat16), u5_ref[...],
                    preferred_element_type=jnp.float32)
    count = jnp.sum(mask.astype(jnp.float32), axis=1, keepdims=True)

    gval = _fiota((R2, NCHUNK_P), 1)
    parts = []
    for s0 in range(0, CIDK, 64):
        sv = np.float32(s0) + _fiota((R2, 64, NCHUNK_P), 1)
        oh = (mask[:, None, :] & (slots[:, None, :] == sv)).astype(jnp.float32)
        parts.append(jnp.sum(oh * gval[:, None, :], axis=2))
    cid = jnp.concatenate(parts, axis=1)

    svec = _fiota((R2, CIDK), 1)
    smod = (jax.lax.broadcasted_iota(jnp.int32, (R2, CIDK), 1) % 9).astype(jnp.float32)
    dummy = np.float32(REAL_CHUNKS) + smod
    cidl = jnp.where(svec < count, cid, dummy)
    row = pid * R2 + _fiota((R2, 1), 0)
    cidf_ref[...] = (cidl + row * NCHUNK).astype(jnp.int32)
    t_ref[...] = jnp.broadcast_to(t, (R2, 128))


def _k2(smx):
    sc = np.zeros((NSUBC, NCHUNK_P), np.float32)
    sc[np.arange(0, NSUBC, SUB), np.arange(NCHUNK)] = 1.0
    u5 = np.triu(np.ones((NCHUNK_P, NCHUNK_P), np.float32), 1).astype(ml_dtypes.bfloat16)
    return pl.pallas_call(
        _k2_body,
        grid=(Q // R2,),
        in_specs=[
            pl.BlockSpec((R2, NSUBC), lambda r: (r, 0)),
            pl.BlockSpec((NSUBC, NCHUNK_P), lambda r: (0, 0)),
            pl.BlockSpec((NCHUNK_P, NCHUNK_P), lambda r: (0, 0)),
        ],
        out_specs=[
            pl.BlockSpec((R2, CIDK), lambda r: (r, 0)),
            pl.BlockSpec((R2, 128), lambda r: (r, 0)),
        ],
        out_shape=[
            jax.ShapeDtypeStruct((Q, CIDK), jnp.int32),
            jax.ShapeDtypeStruct((Q, 128), jnp.float32),
        ],
    )(smx, sc, u5)


R4 = 8
W4 = CIDK * SUB

def _k4_body(smxg_ref, cidf_ref, t_ref, b_ref, u128_ref, eb_ref, sgf_ref, sgl_ref):
    pid = pl.program_id(0)
    g = smxg_ref[...]
    t = t_ref[...][:, 0:1]
    m = g >= t

    slots = _excl_slots(m, b_ref, u128_ref, eb_ref)
    count = jnp.sum(m.astype(jnp.float32), axis=1, keepdims=True)

    row = pid * R4 + _fiota((R4, 1), 0)
    cc = jax.lax.broadcasted_iota(jnp.int32, (CIDK, W4), 0)
    jj = jax.lax.broadcasted_iota(jnp.int32, (CIDK, W4), 1)
    expand = (jj // SUB == cc).astype(jnp.float32)
    cid_local = cidf_ref[...].astype(jnp.float32) - row * NCHUNK
    cid_full = jnp.dot(cid_local, expand, preferred_element_type=jnp.float32,
                       precision=jax.lax.Precision.HIGHEST)
    u = jax.lax.broadcasted_iota(jnp.int32, (R4, W4), 1) % SUB
    sg_full = cid_full * SUB + u.astype(jnp.float32)

    parts = []
    for s0 in range(0, SLOTS, 64):
        sv = np.float32(s0) + _fiota((R4, 64, W4), 1)
        oh = (m[:, None, :] & (slots[:, None, :] == sv)).astype(jnp.float32)
        parts.append(jnp.sum(oh * sg_full[:, None, :], axis=2))
    sg = jnp.concatenate(parts, axis=1)

    svec = _fiota((R4, SLOTS), 1)
    smod = (jax.lax.broadcasted_iota(jnp.int32, (R4, SLOTS), 1) % 128).astype(jnp.float32)
    dummy = np.float32(N // SUB) + smod
    sgl = jnp.where(svec < count, sg, dummy)
    sgl_ref[...] = sgl.astype(jnp.int32)
    sgf_ref[...] = sgl.astype(jnp.int32) + (row.astype(jnp.int32) * NSUBC)


def _k4(smxg, cidf, t):
    b, u128, eb = _scan_consts(W4)
    return pl.pallas_call(
        _k4_body,
        grid=(Q // R4,),
        in_specs=[
            pl.BlockSpec((R4, W4), lambda r: (r, 0)),
            pl.BlockSpec((R4, CIDK), lambda r: (r, 0)),
            pl.BlockSpec((R4, 128), lambda r: (r, 0)),
            pl.BlockSpec((W4, 128), lambda r: (0, 0)),
            pl.BlockSpec((128, 128), lambda r: (0, 0)),
            pl.BlockSpec((128, W4), lambda r: (0, 0)),
        ],
        out_specs=[
            pl.BlockSpec((R4, SLOTS), lambda r: (r, 0)),
            pl.BlockSpec((R4, SLOTS), lambda r: (r, 0)),
        ],
        out_shape=[
            jax.ShapeDtypeStruct((Q, SLOTS), jnp.int32),
            jax.ShapeDtypeStruct((Q, SLOTS), jnp.int32),
        ],
    )(smxg, cidf, t, b, u128, eb)


R6 = 8
W6 = SLOTS * SUB

def _k6a_body(vg_ref, sgl_ref, t_ref, b_ref, u128_ref, eb_ref, e_ref,
              v_ref, c_ref):
    v = vg_ref[...]
    t = t_ref[...][:, 0:1]
    m = v >= t

    slots = _excl_slots(m, b_ref, u128_ref, eb_ref)
    count = jnp.sum(m.astype(jnp.float32), axis=1, keepdims=True)
    sg_full = jnp.dot(sgl_ref[...].astype(jnp.float32), e_ref[...],
                      preferred_element_type=jnp.float32,
                      precision=jax.lax.Precision.HIGHEST)
    u = jax.lax.broadcasted_iota(jnp.int32, (R6, W6), 1) % SUB
    col = sg_full * SUB + u.astype(jnp.float32)

    vparts, cparts = [], []
    for s0 in range(0, SLOTS, 64):
        sv = np.float32(s0) + _fiota((R6, 64, W6), 1)
        oh = (m[:, None, :] & (slots[:, None, :] == sv)).astype(jnp.float32)
        vparts.append(jnp.sum(oh * v[:, None, :], axis=2))
        cparts.append(jnp.sum(oh * col[:, None, :], axis=2))
    v256 = jnp.concatenate(vparts, axis=1)
    c256 = jnp.concatenate(cparts, axis=1)

    svec = _fiota((R6, SLOTS), 1)
    valid = svec < count
    v_ref[...] = jnp.where(valid, v256, NEG)
    c_ref[...] = jnp.where(valid, c256, np.float32(2.0e6) + svec)


def _k6a(vg, sgl, t):
    b, u128, eb = _scan_consts(W6)
    cc = np.arange(SLOTS, dtype=np.int64)
    jj = np.arange(W6, dtype=np.int64)
    ee = (jj[None, :] // SUB == cc[:, None]).astype(np.float32)
    return pl.pallas_call(
        _k6a_body,
        grid=(Q // R6,),
        in_specs=[
            pl.BlockSpec((R6, W6), lambda r: (r, 0)),
            pl.BlockSpec((R6, SLOTS), lambda r: (r, 0)),
            pl.BlockSpec((R6, 128), lambda r: (r, 0)),
            pl.BlockSpec((W6, 128), lambda r: (0, 0)),
            pl.BlockSpec((128, 128), lambda r: (0, 0)),
            pl.BlockSpec((128, W6), lambda r: (0, 0)),
            pl.BlockSpec((SLOTS, W6), lambda r: (0, 0)),
        ],
        out_specs=[
            pl.BlockSpec((R6, SLOTS), lambda r: (r, 0)),
            pl.BlockSpec((R6, SLOTS), lambda r: (r, 0)),
        ],
        out_shape=[
            jax.ShapeDtypeStruct((Q, SLOTS), jnp.float32),
            jax.ShapeDtypeStruct((Q, SLOTS), jnp.float32),
        ],
    )(vg, sgl, t, b, u128, eb, ee)


RB = 128

def _k6b_body(v_ref, c_ref, out_ref):
    v256 = v_ref[...]
    c256 = c_ref[...]
    lane = jax.lax.broadcasted_iota(jnp.int32, (RB, 128), 1)

    def extract(p, carry):
        vc, out = carry
        mx = jnp.max(vc, axis=1, keepdims=True)
        ismx = vc == mx
        cmin = jnp.min(jnp.where(ismx, c256, BIG), axis=1, keepdims=True)
        chosen = ismx & (c256 == cmin)
        out = out + jnp.where(lane == p, cmin, np.float32(0.0))
        vc = jnp.where(chosen, NEG, vc)
        return (vc, out)

    _, out = jax.lax.fori_loop(
        0, KTOP, extract, (v256, jnp.zeros((RB, 128), jnp.float32)))
    out_ref[...] = out.astype(jnp.int32)


def _k6b(v256, c256):
    return pl.pallas_call(
        _k6b_body,
        grid=(Q // RB,),
        in_specs=[
            pl.BlockSpec((RB, SLOTS), lambda r: (r, 0)),
            pl.BlockSpec((RB, SLOTS), lambda r: (r, 0)),
        ],
        out_specs=pl.BlockSpec((RB, 128), lambda r: (r, 0)),
        out_shape=jax.ShapeDtypeStruct((Q, 128), jnp.int32),
    )(v256, c256)


def kernel(image_embs, text_embs):
    text_pad = jnp.pad(text_embs, ((0, NPAD - N), (0, 0)))
    sims, smx = _k1(image_embs, text_pad)

    cidf, t = _k2(smx)

    smx_tab = smx.reshape(Q * NCHUNK, SUB)
    smxg = jnp.take(smx_tab, cidf.reshape(-1), axis=0, mode="clip")
    smxg = smxg.reshape(Q, W4)

    sgf, sgl = _k4(smxg, cidf, t)

    sims_tab = sims.reshape(Q * NSUBC, SUB)
    vg = jnp.take(sims_tab, sgf.reshape(-1), axis=0, mode="clip")
    vg = vg.reshape(Q, W6)

    v256, c256 = _k6a(vg, sgl, t)
    idx = _k6b(v256, c256)
    return idx[:, :KTOP]
